# Initial kernel scaffold; baseline (speedup 1.0000x reference)
import jax
import jax.numpy as jnp
from jax.experimental import pallas as pl


def kernel(X, G, W, b):
    raise NotImplementedError("write your pallas kernel here")


# speedup vs baseline: 1.6609x; 1.6609x over previous
import functools

import jax
import jax.numpy as jnp
from jax import lax
from jax.experimental import pallas as pl
from jax.experimental.pallas import tpu as pltpu
from jax.experimental.pallas import tpu_sc as plsc

NC = 2
NS = 16
L = 16
NW = NC * NS

C = 8


def _tc_matmul(Xp, Wp, n_pad, d_feat, n_cols):
    BN = 512
    BU = 1024

    def body(x_ref, w_ref, y_ref):
        y_ref[...] = jnp.dot(x_ref[...], w_ref[...],
                             preferred_element_type=jnp.float32)

    return pl.pallas_call(
        body,
        grid=(n_pad // BN, n_cols // BU),
        in_specs=[
            pl.BlockSpec((BN, d_feat), lambda i, j: (i, 0)),
            pl.BlockSpec((d_feat, BU), lambda i, j: (0, j)),
        ],
        out_specs=pl.BlockSpec((BN, BU), lambda i, j: (i, j)),
        out_shape=jax.ShapeDtypeStruct((n_pad, n_cols), jnp.float32),
    )(Xp, Wp)


def _sc_gather_reduce(Yr, Gp, b, n_pad, deg, units):
    per_w = n_pad // NW
    n_chunks = per_w // C
    mesh = plsc.VectorSubcoreMesh(core_axis_name="c", subcore_axis_name="s")
    n_acc = units // L

    @functools.partial(
        pl.kernel,
        mesh=mesh,
        out_type=jax.ShapeDtypeStruct((n_pad, units), jnp.float32),
        scratch_types=[
            pltpu.VMEM((C, deg), jnp.int32),
            pltpu.VMEM((2 * C, L, units), jnp.float32),
            pltpu.VMEM((C, units), jnp.float32),
            pltpu.VMEM((units,), jnp.float32),
            pltpu.SemaphoreType.DMA,
        ],
    )
    def k(y_hbm, g_hbm, b_hbm, out_hbm, g_v, rows_v, out_v, b_v, sem):
        wid = lax.axis_index("s") * NC + lax.axis_index("c")
        base = wid * per_w
        pltpu.sync_copy(b_hbm, b_v)

        def chunk_body(i, carry):
            nb = base + i * C
            pltpu.sync_copy(g_hbm.at[pl.ds(nb, C)], g_v)
            copies = []
            for n in range(C):
                for h in range(2):
                    gvec = g_v[n, pl.ds(h * L, L)]
                    idx = gvec * deg + (jnp.arange(L, dtype=jnp.int32) + h * L)
                    copies.append(
                        pltpu.async_copy(y_hbm.at[idx], rows_v.at[2 * n + h], sem))
            for cp in copies:
                cp.wait()

            def node_body(nn, c2):
                accs = [b_v[pl.ds(cc * L, L)] for cc in range(n_acc)]
                for h in range(2):
                    d = 2 * nn + h
                    for r in range(L):
                        for cc in range(n_acc):
                            accs[cc] = accs[cc] + rows_v[d, r, pl.ds(cc * L, L)]
                for cc in range(n_acc):
                    out_v[nn, pl.ds(cc * L, L)] = accs[cc]
                return c2

            lax.fori_loop(0, C, node_body, 0)
            pltpu.sync_copy(out_v, out_hbm.at[pl.ds(nb, C)])
            return carry

        lax.fori_loop(0, n_chunks, chunk_body, 0)

    return k(Yr, Gp, b)


def kernel(X, G, W, b):
    N, D = X.shape
    DEG = G.shape[1]
    U = W.shape[1]
    block = NW * C
    n_pad = -(-N // block) * block

    Wp = W.reshape(DEG, D, U).transpose(1, 0, 2).reshape(D, DEG * U)
    Xp = jnp.pad(X, ((0, n_pad - N), (0, 0)))
    Gp = jnp.pad(G, ((0, n_pad - N), (0, 0)))

    Y = _tc_matmul(Xp, Wp, n_pad, D, DEG * U)
    Yr = Y.reshape(n_pad * DEG, U)

    out = _sc_gather_reduce(Yr, Gp, b, n_pad, DEG, U)
    return out[:N]


# speedup vs baseline: 1.6819x; 1.0126x over previous
import functools

import jax
import jax.numpy as jnp
from jax import lax
from jax.experimental import pallas as pl
from jax.experimental.pallas import tpu as pltpu
from jax.experimental.pallas import tpu_sc as plsc

NC = 2
NS = 16
L = 16
NW = NC * NS

C = 8


def _tc_matmul(Xp, Wp, n_pad, d_feat, n_cols):
    BN = 512
    BU = 1024

    def body(x_ref, w_ref, y_ref):
        y_ref[...] = jnp.dot(x_ref[...], w_ref[...],
                             preferred_element_type=jnp.float32)

    return pl.pallas_call(
        body,
        grid=(n_pad // BN, n_cols // BU),
        in_specs=[
            pl.BlockSpec((BN, d_feat), lambda i, j: (i, 0)),
            pl.BlockSpec((d_feat, BU), lambda i, j: (0, j)),
        ],
        out_specs=pl.BlockSpec((BN, BU), lambda i, j: (i, j)),
        out_shape=jax.ShapeDtypeStruct((n_pad, n_cols), jnp.float32),
    )(Xp, Wp)


def _sc_gather_reduce(Yr, Gp, b, n_pad, deg, units):
    per_w = n_pad // NW
    n_chunks = per_w // C
    mesh = plsc.VectorSubcoreMesh(core_axis_name="c", subcore_axis_name="s")
    n_acc = units // L

    @functools.partial(
        pl.kernel,
        mesh=mesh,
        out_type=jax.ShapeDtypeStruct((n_pad, units), jnp.float32),
        scratch_types=[
            pltpu.VMEM((C, deg), jnp.int32),
            pltpu.VMEM((2 * C, L, units), jnp.float32),
            pltpu.VMEM((C, units), jnp.float32),
            pltpu.VMEM((units,), jnp.float32),
            pltpu.SemaphoreType.DMA,
        ],
    )
    def k(y_hbm, g_hbm, b_hbm, out_hbm, g_v, rows_v, out_v, b_v, sem):
        wid = lax.axis_index("s") * NC + lax.axis_index("c")
        base = wid * per_w
        pltpu.sync_copy(b_hbm, b_v)

        def chunk_body(i, carry):
            nb = base + i * C
            pltpu.sync_copy(g_hbm.at[pl.ds(nb, C)], g_v)
            copies = []
            for n in range(C):
                for h in range(2):
                    gvec = g_v[n, pl.ds(h * L, L)]
                    idx = gvec * deg + (jnp.arange(L, dtype=jnp.int32) + h * L)
                    copies.append(
                        pltpu.async_copy(y_hbm.at[idx], rows_v.at[2 * n + h], sem))
            for cp in copies:
                cp.wait()

            def node_body(nn, c2):
                accs = [b_v[pl.ds(cc * L, L)] for cc in range(n_acc)]
                for h in range(2):
                    d = 2 * nn + h
                    for r in range(L):
                        for cc in range(n_acc):
                            accs[cc] = accs[cc] + rows_v[d, r, pl.ds(cc * L, L)]
                for cc in range(n_acc):
                    out_v[nn, pl.ds(cc * L, L)] = accs[cc]
                return c2

            lax.fori_loop(0, C, node_body, 0)
            pltpu.sync_copy(out_v, out_hbm.at[pl.ds(nb, C)])
            return carry

        lax.fori_loop(0, n_chunks, chunk_body, 0)

    return k(Yr, Gp, b)


def kernel(X, G, W, b):
    N, D = X.shape
    DEG = G.shape[1]
    U = W.shape[1]
    block = NW * C
    n_pad = -(-N // block) * block

    Wp = W.reshape(DEG, D, U).transpose(1, 0, 2).reshape(D, DEG * U)
    Xp = jnp.pad(X, ((0, n_pad - N), (0, 0)))
    Gp = jnp.pad(G, ((0, n_pad - N), (0, 0)))

    Y = _tc_matmul(Xp.astype(jnp.bfloat16), Wp.astype(jnp.bfloat16),
                   n_pad, D, DEG * U)
    Yr = Y.reshape(n_pad * DEG, U)

    out = _sc_gather_reduce(Yr, Gp, b, n_pad, DEG, U)
    return out[:N]
